# baseline (speedup 1.0000x reference)
import jax
import jax.numpy as jnp
from jax.experimental import pallas as pl

NUM_LAYERS = 4
D = 1024
OUT = 64
TM = 512


def _moe_kernel(x_ref, wr_ref, wl_ref, bl_ref, we_ref, be_ref, out_ref):
    x = x_ref[...]
    logits = jnp.dot(x, wr_ref[...], preferred_element_type=jnp.float32)
    g = jax.nn.softmax(logits, axis=-1)
    heads = [g[:, 0:1], g[:, 1:2], g[:, 2:3]]
    g3 = g[:, 3:4]
    sufs = [g[:, 1:2] + g[:, 2:3] + g3, g[:, 2:3] + g3, g3]

    xb = x.astype(jnp.bfloat16)
    cur = x
    curb = xb
    acc = jnp.zeros((x.shape[0], OUT), dtype=jnp.float32)
    keep = jnp.ones((x.shape[0], 1), dtype=jnp.bool_)
    for d in range(NUM_LAYERS):
        h = jnp.dot(curb, wl_ref[d, :D, :], preferred_element_type=jnp.float32)
        h = h + jnp.dot(xb, wl_ref[d, D:, :], preferred_element_type=jnp.float32)
        h = jnp.maximum(h + bl_ref[d:d + 1, :], 0.0)
        cur = cur + h
        curb = cur.astype(jnp.bfloat16)
        p = jnp.dot(curb, we_ref[d], preferred_element_type=jnp.float32)
        p = p + be_ref[d:d + 1, :]
        if d < NUM_LAYERS - 1:
            raw0, raw1 = heads[d], sufs[d]
            denom = jnp.abs(raw0) + jnp.abs(raw1)
            mask_exit = (raw0 / denom) > 0.0
            mask_enter = (raw1 / denom) > 0.0
            acc = acc + jnp.where(jnp.logical_and(keep, mask_exit), p, 0.0)
            keep = jnp.logical_and(keep, mask_enter)
        else:
            acc = acc + jnp.where(keep, p, 0.0)
    out_ref[...] = acc


def kernel(inputs, Wr, Wl, bl, We, be):
    n_tokens = inputs.shape[0]
    Wl = Wl.astype(jnp.bfloat16)
    We = We.astype(jnp.bfloat16)
    return pl.pallas_call(
        _moe_kernel,
        grid=(n_tokens // TM,),
        in_specs=[
            pl.BlockSpec((TM, D), lambda i: (i, 0)),
            pl.BlockSpec((D, NUM_LAYERS), lambda i: (0, 0)),
            pl.BlockSpec((NUM_LAYERS, 2 * D, D), lambda i: (0, 0, 0)),
            pl.BlockSpec((NUM_LAYERS, D), lambda i: (0, 0)),
            pl.BlockSpec((NUM_LAYERS, D, OUT), lambda i: (0, 0, 0)),
            pl.BlockSpec((NUM_LAYERS, OUT), lambda i: (0, 0)),
        ],
        out_specs=pl.BlockSpec((TM, OUT), lambda i: (i, 0)),
        out_shape=jax.ShapeDtypeStruct((n_tokens, OUT), jnp.float32),
    )(inputs, Wr, Wl, bl, We, be)


# speedup vs baseline: 1.1211x; 1.1211x over previous
import jax
import jax.numpy as jnp
from jax.experimental import pallas as pl

NUM_LAYERS = 4
D = 1024
OUT = 64
TM = 512


def _moe_kernel(x_ref, wr_ref, wl_ref, bl_ref, we_ref, be_ref, out_ref):
    x = x_ref[...]
    logits = jnp.dot(x, wr_ref[...], preferred_element_type=jnp.float32)
    g = jax.nn.softmax(logits, axis=-1)
    heads = [g[:, 0:1], g[:, 1:2], g[:, 2:3]]
    g3 = g[:, 3:4]
    sufs = [g[:, 1:2] + g[:, 2:3] + g3, g[:, 2:3] + g3, g3]

    cur = x
    acc = jnp.zeros((x.shape[0], OUT), dtype=jnp.float32)
    keep = jnp.ones((x.shape[0], 1), dtype=jnp.bool_)
    for d in range(NUM_LAYERS):
        h = jnp.dot(cur, wl_ref[d, :D, :], preferred_element_type=jnp.float32)
        h = h + jnp.dot(x, wl_ref[d, D:, :], preferred_element_type=jnp.float32)
        h = jnp.maximum(h + bl_ref[d:d + 1, :], 0.0)
        cur = cur + h
        p = jnp.dot(cur, we_ref[d], preferred_element_type=jnp.float32)
        p = p + be_ref[d:d + 1, :]
        if d < NUM_LAYERS - 1:
            raw0, raw1 = heads[d], sufs[d]
            denom = jnp.abs(raw0) + jnp.abs(raw1)
            mask_exit = (raw0 / denom) > 0.0
            mask_enter = (raw1 / denom) > 0.0
            acc = acc + jnp.where(jnp.logical_and(keep, mask_exit), p, 0.0)
            keep = jnp.logical_and(keep, mask_enter)
        else:
            acc = acc + jnp.where(keep, p, 0.0)
    out_ref[...] = acc


def kernel(inputs, Wr, Wl, bl, We, be):
    n_tokens = inputs.shape[0]
    return pl.pallas_call(
        _moe_kernel,
        grid=(n_tokens // TM,),
        in_specs=[
            pl.BlockSpec((TM, D), lambda i: (i, 0)),
            pl.BlockSpec((D, NUM_LAYERS), lambda i: (0, 0)),
            pl.BlockSpec((NUM_LAYERS, 2 * D, D), lambda i: (0, 0, 0)),
            pl.BlockSpec((NUM_LAYERS, D), lambda i: (0, 0)),
            pl.BlockSpec((NUM_LAYERS, D, OUT), lambda i: (0, 0, 0)),
            pl.BlockSpec((NUM_LAYERS, OUT), lambda i: (0, 0)),
        ],
        out_specs=pl.BlockSpec((TM, OUT), lambda i: (i, 0)),
        out_shape=jax.ShapeDtypeStruct((n_tokens, OUT), jnp.float32),
    )(inputs, Wr, Wl, bl, We, be)


# speedup vs baseline: 1.1432x; 1.0197x over previous
import jax
import jax.numpy as jnp
from jax.experimental import pallas as pl
from jax.experimental.pallas import tpu as pltpu

NUM_LAYERS = 4
D = 1024
OUT = 64
TM = 512


def _moe_kernel(x_ref, wr_ref, wl_ref, bl_ref, we_ref, be_ref, out_ref,
                w0_ref):
    x = x_ref[...]
    logits = jnp.dot(x, wr_ref[...], preferred_element_type=jnp.float32)
    g = jax.nn.softmax(logits, axis=-1)
    heads = [g[:, 0:1], g[:, 1:2], g[:, 2:3]]
    g3 = g[:, 3:4]
    sufs = [g[:, 1:2] + g[:, 2:3] + g3, g[:, 2:3] + g3, g3]

    @pl.when(pl.program_id(0) == 0)
    def _build_w0():
        w0_ref[...] = wl_ref[0, :D, :] + wl_ref[0, D:, :]

    cur = x
    acc = jnp.zeros((x.shape[0], OUT), dtype=jnp.float32)
    keep = jnp.ones((x.shape[0], 1), dtype=jnp.bool_)
    for d in range(NUM_LAYERS):
        if d == 0:
            h = jnp.dot(x, w0_ref[...], preferred_element_type=jnp.float32)
        else:
            h = jnp.dot(cur, wl_ref[d, :D, :],
                        preferred_element_type=jnp.float32)
            h = h + jnp.dot(x, wl_ref[d, D:, :],
                            preferred_element_type=jnp.float32)
        h = jnp.maximum(h + bl_ref[d:d + 1, :], 0.0)
        cur = cur + h
        p = jnp.dot(cur, we_ref[d], preferred_element_type=jnp.float32)
        p = p + be_ref[d:d + 1, :]
        if d < NUM_LAYERS - 1:
            raw0, raw1 = heads[d], sufs[d]
            denom = jnp.abs(raw0) + jnp.abs(raw1)
            mask_exit = (raw0 / denom) > 0.0
            mask_enter = (raw1 / denom) > 0.0
            acc = acc + jnp.where(jnp.logical_and(keep, mask_exit), p, 0.0)
            keep = jnp.logical_and(keep, mask_enter)
        else:
            acc = acc + jnp.where(keep, p, 0.0)
    out_ref[...] = acc


def kernel(inputs, Wr, Wl, bl, We, be):
    n_tokens = inputs.shape[0]
    return pl.pallas_call(
        _moe_kernel,
        grid=(n_tokens // TM,),
        in_specs=[
            pl.BlockSpec((TM, D), lambda i: (i, 0)),
            pl.BlockSpec((D, NUM_LAYERS), lambda i: (0, 0)),
            pl.BlockSpec((NUM_LAYERS, 2 * D, D), lambda i: (0, 0, 0)),
            pl.BlockSpec((NUM_LAYERS, D), lambda i: (0, 0)),
            pl.BlockSpec((NUM_LAYERS, D, OUT), lambda i: (0, 0, 0)),
            pl.BlockSpec((NUM_LAYERS, OUT), lambda i: (0, 0)),
        ],
        out_specs=pl.BlockSpec((TM, OUT), lambda i: (i, 0)),
        out_shape=jax.ShapeDtypeStruct((n_tokens, OUT), jnp.float32),
        scratch_shapes=[pltpu.VMEM((D, D), jnp.float32)],
    )(inputs, Wr, Wl, bl, We, be)
